# baseline (speedup 1.0000x reference)
import functools

import jax
import jax.numpy as jnp
from jax.experimental import pallas as pl
from jax.experimental.pallas import tpu as pltpu
from jax.experimental.pallas import tpu_sc as plsc

NUM_EXPERT = 8
TOP_K = 2
D_MODEL = 768
D_HIDDEN = 768
N_TOKENS = 4096

TOKEN_TILE = 512


def _moe_body(x_ref, wg_ref, bg_ref, w1_ref, b1_ref, w2_ref, b2_ref, out_ref):
    x = x_ref[...]

    logits = (
        jnp.dot(x, wg_ref[...], preferred_element_type=jnp.float32)
        + bg_ref[...]
    )
    e_iota = jax.lax.broadcasted_iota(jnp.int32, logits.shape, 1)
    i1 = jnp.argmax(logits, axis=-1)[:, None]
    v1 = jnp.max(logits, axis=-1, keepdims=True)
    masked = jnp.where(e_iota == i1, -jnp.inf, logits)
    i2 = jnp.argmax(masked, axis=-1)[:, None]
    v2 = jnp.max(masked, axis=-1, keepdims=True)
    d = jnp.exp(v2 - v1)
    s1 = 1.0 / (1.0 + d)
    s2 = d / (1.0 + d)
    w = jnp.where(e_iota == i1, s1, 0.0) + jnp.where(e_iota == i2, s2, 0.0)

    acc = jnp.zeros(x.shape, dtype=jnp.float32)
    xb = x.astype(jnp.bfloat16)
    for e in range(NUM_EXPERT):
        h = jnp.dot(xb, w1_ref[e].astype(jnp.bfloat16),
                    preferred_element_type=jnp.float32)
        h = jnp.maximum(h + b1_ref[e], 0.0)
        y = jnp.dot(h.astype(jnp.bfloat16), w2_ref[e].astype(jnp.bfloat16),
                    preferred_element_type=jnp.float32)
        y = y + b2_ref[e]
        acc = acc + w[:, e][:, None] * y
    out_ref[...] = acc


_GATHER_ROWS = 10240
_NW = 32
_BPW = _GATHER_ROWS // _NW
_CHUNK = 64
_NCHUNK = _BPW // _CHUNK


def _make_sc_gather(d):
    mesh = plsc.VectorSubcoreMesh(core_axis_name="c", subcore_axis_name="s")

    @functools.partial(
        pl.kernel,
        mesh=mesh,
        out_type=jax.ShapeDtypeStruct((_GATHER_ROWS, d), jnp.float32),
        scratch_types=[
            pltpu.VMEM((_BPW,), jnp.int32),
            pltpu.VMEM((_CHUNK, d), jnp.float32),
            pltpu.SemaphoreType.DMA,
        ],
    )
    def gather_k(table_hbm, idx_hbm, out_hbm, idx_v, rows_v, sem):
        wid = jax.lax.axis_index("s") * 2 + jax.lax.axis_index("c")
        base = wid * _BPW
        pltpu.sync_copy(idx_hbm.at[pl.ds(base, _BPW)], idx_v)
        for c in range(_NCHUNK):
            pltpu.async_copy(
                table_hbm.at[idx_v.at[pl.ds(c * _CHUNK, _CHUNK)]], rows_v, sem
            ).wait()
            pltpu.sync_copy(rows_v, out_hbm.at[pl.ds(base + c * _CHUNK, _CHUNK)])

    return gather_k


@jax.jit
def kernel(moe_inp, Wg, bg, W1, b1, W2, b2):
    n = moe_inp.shape[0]
    grid = (n // TOKEN_TILE,)
    bg2 = bg.reshape(1, NUM_EXPERT)
    dense_out = pl.pallas_call(
        _moe_body,
        grid=grid,
        in_specs=[
            pl.BlockSpec((TOKEN_TILE, D_MODEL), lambda i: (i, 0)),
            pl.BlockSpec((D_MODEL, NUM_EXPERT), lambda i: (0, 0)),
            pl.BlockSpec((1, NUM_EXPERT), lambda i: (0, 0)),
            pl.BlockSpec((NUM_EXPERT, D_MODEL, D_HIDDEN), lambda i: (0, 0, 0)),
            pl.BlockSpec((NUM_EXPERT, D_HIDDEN), lambda i: (0, 0)),
            pl.BlockSpec((NUM_EXPERT, D_HIDDEN, D_MODEL), lambda i: (0, 0, 0)),
            pl.BlockSpec((NUM_EXPERT, D_MODEL), lambda i: (0, 0)),
        ],
        out_specs=pl.BlockSpec((TOKEN_TILE, D_MODEL), lambda i: (i, 0)),
        out_shape=jax.ShapeDtypeStruct((n, D_MODEL), jnp.float32),
    )(moe_inp, Wg, bg2, W1, b1, W2, b2)
    idx = (jnp.arange(_GATHER_ROWS, dtype=jnp.int32) % n)
    gathered = _make_sc_gather(D_MODEL)(moe_inp, idx)
    return dense_out + (gathered[:n] - moe_inp)


# speedup vs baseline: 1.2275x; 1.2275x over previous
import functools

import jax
import jax.numpy as jnp
from jax.experimental import pallas as pl
from jax.experimental.pallas import tpu as pltpu
from jax.experimental.pallas import tpu_sc as plsc

NUM_EXPERT = 8
TOP_K = 2
D_MODEL = 768
D_HIDDEN = 768
N_TOKENS = 4096

TOKEN_TILE = 4096


def _moe_body(x_ref, wg_ref, bg_ref, w1_ref, b1_ref, w2_ref, b2_ref, out_ref,
              w_scratch):
    e = pl.program_id(1)
    x = x_ref[...]

    @pl.when(e == 0)
    def _gate():
        logits = (
            jnp.dot(x, wg_ref[...], preferred_element_type=jnp.float32)
            + bg_ref[...]
        )
        e_iota = jax.lax.broadcasted_iota(jnp.int32, logits.shape, 1)
        i1 = jnp.argmax(logits, axis=-1)[:, None]
        v1 = jnp.max(logits, axis=-1, keepdims=True)
        masked = jnp.where(e_iota == i1, -jnp.inf, logits)
        i2 = jnp.argmax(masked, axis=-1)[:, None]
        v2 = jnp.max(masked, axis=-1, keepdims=True)
        d = jnp.exp(v2 - v1)
        s1 = 1.0 / (1.0 + d)
        s2 = d / (1.0 + d)
        w_scratch[...] = (
            jnp.where(e_iota == i1, s1, 0.0) + jnp.where(e_iota == i2, s2, 0.0)
        )

    xb = x.astype(jnp.bfloat16)
    h = jnp.dot(xb, w1_ref[0].astype(jnp.bfloat16),
                preferred_element_type=jnp.float32)
    h = jnp.maximum(h + b1_ref[0, 0], 0.0)
    y = jnp.dot(h.astype(jnp.bfloat16), w2_ref[0].astype(jnp.bfloat16),
                preferred_element_type=jnp.float32)
    y = y + b2_ref[0, 0]

    w = w_scratch[...]
    e_iota = jax.lax.broadcasted_iota(jnp.int32, w.shape, 1)
    w_col = jnp.sum(jnp.where(e_iota == e, w, 0.0), axis=1)[:, None]
    contrib = w_col * y

    @pl.when(e == 0)
    def _init():
        out_ref[...] = contrib

    @pl.when(e > 0)
    def _acc():
        out_ref[...] = out_ref[...] + contrib


@jax.jit
def kernel(moe_inp, Wg, bg, W1, b1, W2, b2):
    n = moe_inp.shape[0]
    grid = (n // TOKEN_TILE, NUM_EXPERT)
    bg2 = bg.reshape(1, NUM_EXPERT)
    return pl.pallas_call(
        _moe_body,
        grid=grid,
        in_specs=[
            pl.BlockSpec((TOKEN_TILE, D_MODEL), lambda i, e: (i, 0)),
            pl.BlockSpec((D_MODEL, NUM_EXPERT), lambda i, e: (0, 0)),
            pl.BlockSpec((1, NUM_EXPERT), lambda i, e: (0, 0)),
            pl.BlockSpec((1, D_MODEL, D_HIDDEN), lambda i, e: (e, 0, 0)),
            pl.BlockSpec((1, 1, D_HIDDEN), lambda i, e: (e, 0, 0)),
            pl.BlockSpec((1, D_HIDDEN, D_MODEL), lambda i, e: (e, 0, 0)),
            pl.BlockSpec((1, 1, D_MODEL), lambda i, e: (e, 0, 0)),
        ],
        out_specs=pl.BlockSpec((TOKEN_TILE, D_MODEL), lambda i, e: (i, 0)),
        out_shape=jax.ShapeDtypeStruct((n, D_MODEL), jnp.float32),
        scratch_shapes=[pltpu.VMEM((TOKEN_TILE, NUM_EXPERT), jnp.float32)],
    )(moe_inp, Wg, bg2, W1,
      b1.reshape(NUM_EXPERT, 1, D_HIDDEN),
      W2,
      b2.reshape(NUM_EXPERT, 1, D_MODEL))


# speedup vs baseline: 1.4711x; 1.1985x over previous
import functools

import jax
import jax.numpy as jnp
from jax.experimental import pallas as pl
from jax.experimental.pallas import tpu as pltpu

NUM_EXPERT = 8
TOP_K = 2
D_MODEL = 768
D_HIDDEN = 768
N_TOKENS = 4096

TOKEN_TILE = 1024


def _moe_body(x_ref, wg_ref, bg_ref, w1_ref, b1_ref, w2_ref, b2_ref, out_ref):
    x = x_ref[...]

    logits = (
        jnp.dot(x, wg_ref[...], preferred_element_type=jnp.float32)
        + bg_ref[...]
    )
    e_iota = jax.lax.broadcasted_iota(jnp.int32, logits.shape, 1)
    i1 = jnp.argmax(logits, axis=-1)[:, None]
    v1 = jnp.max(logits, axis=-1, keepdims=True)
    masked = jnp.where(e_iota == i1, -jnp.inf, logits)
    i2 = jnp.argmax(masked, axis=-1)[:, None]
    v2 = jnp.max(masked, axis=-1, keepdims=True)
    d = jnp.exp(v2 - v1)
    s1 = 1.0 / (1.0 + d)
    s2 = d / (1.0 + d)
    w = jnp.where(e_iota == i1, s1, 0.0) + jnp.where(e_iota == i2, s2, 0.0)

    acc = jnp.zeros(x.shape, dtype=jnp.float32)
    xb = x.astype(jnp.bfloat16)
    for e in range(NUM_EXPERT):
        h = jnp.dot(xb, w1_ref[e].astype(jnp.bfloat16),
                    preferred_element_type=jnp.float32)
        h = jnp.maximum(h + b1_ref[e], 0.0)
        y = jnp.dot(h.astype(jnp.bfloat16), w2_ref[e].astype(jnp.bfloat16),
                    preferred_element_type=jnp.float32)
        y = y + b2_ref[e]
        acc = acc + w[:, e][:, None] * y
    out_ref[...] = acc


@jax.jit
def kernel(moe_inp, Wg, bg, W1, b1, W2, b2):
    n = moe_inp.shape[0]
    grid = (n // TOKEN_TILE,)
    bg2 = bg.reshape(1, NUM_EXPERT)
    return pl.pallas_call(
        _moe_body,
        grid=grid,
        in_specs=[
            pl.BlockSpec((TOKEN_TILE, D_MODEL), lambda i: (i, 0)),
            pl.BlockSpec((D_MODEL, NUM_EXPERT), lambda i: (0, 0)),
            pl.BlockSpec((1, NUM_EXPERT), lambda i: (0, 0)),
            pl.BlockSpec((NUM_EXPERT, D_MODEL, D_HIDDEN), lambda i: (0, 0, 0)),
            pl.BlockSpec((NUM_EXPERT, D_HIDDEN), lambda i: (0, 0)),
            pl.BlockSpec((NUM_EXPERT, D_HIDDEN, D_MODEL), lambda i: (0, 0, 0)),
            pl.BlockSpec((NUM_EXPERT, D_MODEL), lambda i: (0, 0)),
        ],
        out_specs=pl.BlockSpec((TOKEN_TILE, D_MODEL), lambda i: (i, 0)),
        out_shape=jax.ShapeDtypeStruct((n, D_MODEL), jnp.float32),
    )(moe_inp, Wg, bg2, W1, b1, W2, b2)
